# baseline (speedup 1.0000x reference)
import functools

import jax
import jax.numpy as jnp
from jax import lax
from jax.experimental import pallas as pl
from jax.experimental.pallas import tpu as pltpu
from jax.experimental.pallas import tpu_sc as plsc

_POSE_NUM = 100000
_EMBED_DIM = 6
_BATCH = 16384

_info = plsc.get_sparse_core_info()
_NC, _NS = _info.num_cores, _info.num_subcores
_NW = _NC * _NS
_B_PER_W = _BATCH // _NW
_E_PER_W = _B_PER_W * _EMBED_DIM
_LANES = 16
_CHUNKS = _B_PER_W // _LANES


def _make_gather():
    mesh = plsc.VectorSubcoreMesh(core_axis_name="c", subcore_axis_name="s")

    @functools.partial(
        pl.kernel,
        mesh=mesh,
        out_type=jax.ShapeDtypeStruct((_BATCH * _EMBED_DIM,), jnp.float32),
        scratch_types=[
            pltpu.VMEM((_B_PER_W,), jnp.int32),
            pltpu.VMEM((_E_PER_W,), jnp.int32),
            pltpu.VMEM((_E_PER_W,), jnp.float32),
            pltpu.SemaphoreType.DMA,
        ],
        compiler_params=pltpu.CompilerParams(
            use_tc_tiling_on_sc=False, needs_layout_passes=False
        ),
    )
    def gather_kernel(
        idx_hbm, table2d_hbm, flat_tab_hbm, out_hbm, idx_v, fidx_v, vals_v, sem
    ):
        del table2d_hbm
        wid = lax.axis_index("s") * _NC + lax.axis_index("c")
        base = wid * _B_PER_W
        pltpu.sync_copy(idx_hbm.at[pl.ds(base, _B_PER_W)], idx_v)

        lane = lax.iota(jnp.int32, _LANES)
        for c in range(_CHUNKS):
            row_idx = idx_v[pl.ds(c * _LANES, _LANES)]
            elem0 = row_idx * _EMBED_DIM
            dst0 = lane * _EMBED_DIM + (c * _LANES * _EMBED_DIM)
            for j in range(_EMBED_DIM):
                plsc.store_scatter(fidx_v, [dst0 + j], elem0 + j)

        pltpu.async_copy(flat_tab_hbm.at[fidx_v], vals_v, sem).wait()
        pltpu.sync_copy(vals_v, out_hbm.at[pl.ds(base * _EMBED_DIM, _E_PER_W)])

    return gather_kernel


_gather = _make_gather()


def kernel(indices, table):
    flat = _gather(indices.astype(jnp.int32), table, table.reshape(-1))
    return flat.reshape(_BATCH, _EMBED_DIM)


# speedup vs baseline: 1.4041x; 1.4041x over previous
import functools

import jax
import jax.numpy as jnp
from jax import lax
from jax.experimental import pallas as pl
from jax.experimental.pallas import tpu as pltpu
from jax.experimental.pallas import tpu_sc as plsc

_POSE_NUM = 100000
_EMBED_DIM = 6
_ROW_PAD = 8
_BATCH = 16384

_info = plsc.get_sparse_core_info()
_NC, _NS = _info.num_cores, _info.num_subcores
_NW = _NC * _NS
_B_PER_W = _BATCH // _NW


def _make_gather():
    mesh = plsc.VectorSubcoreMesh(core_axis_name="c", subcore_axis_name="s")

    @functools.partial(
        pl.kernel,
        mesh=mesh,
        out_type=jax.ShapeDtypeStruct((_BATCH, _ROW_PAD), jnp.float32),
        scratch_types=[
            pltpu.VMEM((_B_PER_W,), jnp.int32),
            pltpu.VMEM((_B_PER_W, _ROW_PAD), jnp.float32),
            pltpu.SemaphoreType.DMA,
        ],
        compiler_params=pltpu.CompilerParams(
            use_tc_tiling_on_sc=False, needs_layout_passes=False
        ),
    )
    def gather_kernel(idx_hbm, table_hbm, out_hbm, idx_v, rows_v, sem):
        wid = lax.axis_index("s") * _NC + lax.axis_index("c")
        base = wid * _B_PER_W
        pltpu.sync_copy(idx_hbm.at[pl.ds(base, _B_PER_W)], idx_v)
        pltpu.async_copy(table_hbm.at[idx_v], rows_v, sem).wait()
        pltpu.sync_copy(rows_v, out_hbm.at[pl.ds(base, _B_PER_W)])

    return gather_kernel


_gather = _make_gather()


def kernel(indices, table):
    padded = jnp.pad(table, ((0, 0), (0, _ROW_PAD - _EMBED_DIM)))
    out8 = _gather(indices.astype(jnp.int32), padded)
    return out8[:, :_EMBED_DIM]


# speedup vs baseline: 6.1445x; 4.3762x over previous
import functools

import jax
import jax.numpy as jnp
from jax import lax
from jax.experimental import pallas as pl
from jax.experimental.pallas import tpu as pltpu
from jax.experimental.pallas import tpu_sc as plsc

_POSE_NUM = 100000
_EMBED_DIM = 6
_BATCH = 16384

_info = plsc.get_sparse_core_info()
_NC, _NS = _info.num_cores, _info.num_subcores
_NW = _NC * _NS
_B_PER_W = _BATCH // _NW
_E_PER_W = _B_PER_W * _EMBED_DIM
_LANES = 16
_CHUNKS = _B_PER_W // _LANES


def _make_gather():
    mesh = plsc.VectorSubcoreMesh(core_axis_name="c", subcore_axis_name="s")

    @functools.partial(
        pl.kernel,
        mesh=mesh,
        out_type=jax.ShapeDtypeStruct((_BATCH * _EMBED_DIM,), jnp.float32),
        scratch_types=[
            pltpu.VMEM((_B_PER_W,), jnp.int32),
            pltpu.VMEM((_E_PER_W,), jnp.int32),
            pltpu.VMEM((_E_PER_W,), jnp.float32),
            pltpu.SemaphoreType.DMA,
        ],
        compiler_params=pltpu.CompilerParams(
            use_tc_tiling_on_sc=False, needs_layout_passes=False
        ),
    )
    def gather_kernel(idx_hbm, tab_cm_hbm, out_hbm, idx_v, fidx_v, vals_v, sem):
        wid = lax.axis_index("s") * _NC + lax.axis_index("c")
        base = wid * _B_PER_W
        pltpu.sync_copy(idx_hbm.at[pl.ds(base, _B_PER_W)], idx_v)

        for c in range(_CHUNKS):
            row_idx = idx_v[pl.ds(c * _LANES, _LANES)]
            for j in range(_EMBED_DIM):
                fidx_v[pl.ds(j * _B_PER_W + c * _LANES, _LANES)] = (
                    row_idx + j * _POSE_NUM
                )

        pltpu.async_copy(tab_cm_hbm.at[fidx_v], vals_v, sem).wait()
        for j in range(_EMBED_DIM):
            pltpu.sync_copy(
                vals_v.at[pl.ds(j * _B_PER_W, _B_PER_W)],
                out_hbm.at[pl.ds(j * _BATCH + base, _B_PER_W)],
            )

    return gather_kernel


_gather = _make_gather()


def kernel(indices, table):
    flat_cm = table.T.reshape(-1)
    out_cm = _gather(indices.astype(jnp.int32), flat_cm)
    return out_cm.reshape(_EMBED_DIM, _BATCH).T


# speedup vs baseline: 6.2375x; 1.0151x over previous
import functools

import jax
import jax.numpy as jnp
from jax import lax
from jax.experimental import pallas as pl
from jax.experimental.pallas import tpu as pltpu
from jax.experimental.pallas import tpu_sc as plsc

_POSE_NUM = 100000
_EMBED_DIM = 6
_BATCH = 16384

_info = plsc.get_sparse_core_info()
_NC, _NS = _info.num_cores, _info.num_subcores
_NW = _NC * _NS
_B_PER_W = _BATCH // _NW
_E_PER_W = _B_PER_W * _EMBED_DIM


def _make_gather():
    mesh = plsc.VectorSubcoreMesh(core_axis_name="c", subcore_axis_name="s")

    @functools.partial(
        pl.kernel,
        mesh=mesh,
        out_type=jax.ShapeDtypeStruct((_BATCH * _EMBED_DIM,), jnp.float32),
        scratch_types=[
            pltpu.VMEM((_B_PER_W,), jnp.int32),
            pltpu.VMEM((_E_PER_W,), jnp.float32),
            pltpu.SemaphoreType.DMA,
            pltpu.SemaphoreType.DMA,
        ],
        compiler_params=pltpu.CompilerParams(
            use_tc_tiling_on_sc=False, needs_layout_passes=False
        ),
    )
    def gather_kernel(idx_hbm, tab_cm_hbm, out_hbm, idx_v, vals_v, gsem, osem):
        wid = lax.axis_index("s") * _NC + lax.axis_index("c")
        base = wid * _B_PER_W
        pltpu.sync_copy(idx_hbm.at[pl.ds(base, _B_PER_W)], idx_v)

        gathers = []
        for j in range(_EMBED_DIM):
            col = tab_cm_hbm.at[pl.ds(j * _POSE_NUM, _POSE_NUM)]
            gathers.append(
                pltpu.async_copy(
                    col.at[idx_v],
                    vals_v.at[pl.ds(j * _B_PER_W, _B_PER_W)],
                    gsem,
                )
            )
        for g in gathers:
            g.wait()

        stores = []
        for j in range(_EMBED_DIM):
            stores.append(
                pltpu.async_copy(
                    vals_v.at[pl.ds(j * _B_PER_W, _B_PER_W)],
                    out_hbm.at[pl.ds(j * _BATCH + base, _B_PER_W)],
                    osem,
                )
            )
        for s in stores:
            s.wait()

    return gather_kernel


_gather = _make_gather()


def kernel(indices, table):
    flat_cm = table.T.reshape(-1)
    out_cm = _gather(indices.astype(jnp.int32), flat_cm)
    return out_cm.reshape(_EMBED_DIM, _BATCH).T


# speedup vs baseline: 6.2748x; 1.0060x over previous
import functools

import jax
import jax.numpy as jnp
from jax import lax
from jax.experimental import pallas as pl
from jax.experimental.pallas import tpu as pltpu
from jax.experimental.pallas import tpu_sc as plsc

_POSE_NUM = 100000
_EMBED_DIM = 6
_BATCH = 16384

_info = plsc.get_sparse_core_info()
_NC, _NS = _info.num_cores, _info.num_subcores
_NW = _NC * _NS
_B_PER_W = _BATCH // _NW
_E_PER_W = _B_PER_W * _EMBED_DIM


def _make_gather():
    mesh = plsc.VectorSubcoreMesh(core_axis_name="c", subcore_axis_name="s")

    @functools.partial(
        pl.kernel,
        mesh=mesh,
        out_type=jax.ShapeDtypeStruct((_BATCH * _EMBED_DIM,), jnp.float32),
        scratch_types=[
            pltpu.VMEM((_B_PER_W,), jnp.int32),
            pltpu.VMEM((_E_PER_W,), jnp.float32),
            pltpu.SemaphoreType.DMA,
            pltpu.SemaphoreType.DMA,
        ],
        compiler_params=pltpu.CompilerParams(
            use_tc_tiling_on_sc=False, needs_layout_passes=False
        ),
    )
    def gather_kernel(idx_hbm, tab_cm_hbm, out_hbm, idx_v, vals_v, gsem, osem):
        wid = lax.axis_index("s") * _NC + lax.axis_index("c")
        base = wid * _B_PER_W
        pltpu.sync_copy(idx_hbm.at[pl.ds(base, _B_PER_W)], idx_v)

        gathers = []
        for j in range(_EMBED_DIM):
            col = tab_cm_hbm.at[pl.ds(j * _POSE_NUM, _POSE_NUM)]
            gathers.append(
                pltpu.async_copy(
                    col.at[idx_v],
                    vals_v.at[pl.ds(j * _B_PER_W, _B_PER_W)],
                    gsem,
                )
            )
        stores = []
        for j in range(_EMBED_DIM):
            gathers[j].wait()
            stores.append(
                pltpu.async_copy(
                    vals_v.at[pl.ds(j * _B_PER_W, _B_PER_W)],
                    out_hbm.at[pl.ds(j * _BATCH + base, _B_PER_W)],
                    osem,
                )
            )
        for s in stores:
            s.wait()

    return gather_kernel


_gather = _make_gather()


def kernel(indices, table):
    flat_cm = table.T.reshape(-1)
    out_cm = _gather(indices.astype(jnp.int32), flat_cm)
    return out_cm.reshape(_EMBED_DIM, _BATCH).T
